# baseline (speedup 1.0000x reference)
import functools

import jax
import jax.numpy as jnp
from jax import lax
from jax.experimental import pallas as pl
from jax.experimental.pallas import tpu as pltpu
from jax.experimental.pallas import tpu_sc as plsc

N = 10000
NP = 10240
F = 111
D = 256
HALF = 128
DN = 32
DD = 64
E = 80000

NC = 2
NS = 16
CHUNK = 128
NCHUNK = 40
EPAD = NS * NCHUNK * CHUNK
TRASH = 10200
RPT = NP // NS

_mesh = functools.partial(
    plsc.VectorSubcoreMesh, core_axis_name="c", subcore_axis_name="s",
    num_cores=NC, num_subcores=NS)


def _deg_body(dst_hbm, zeros_hbm, ones_hbm, out_hbm, idx_v, ones_v, acc, sem):
    c = lax.axis_index("c")
    t = lax.axis_index("s")
    pltpu.sync_copy(zeros_hbm, acc.at[pl.ds(t * RPT, RPT)])
    pltpu.sync_copy(ones_hbm, ones_v)
    pltpu.sync_copy(dst_hbm.at[c, t], idx_v)
    plsc.subcore_barrier()

    def body(j, carry):
        pltpu.sync_copy(ones_v, acc.at[idx_v.at[j]], add=True)
        return carry

    lax.fori_loop(0, NCHUNK, body, 0)
    plsc.subcore_barrier()
    pltpu.sync_copy(acc.at[pl.ds(t * RPT, RPT)],
                    out_hbm.at[c, pl.ds(t * RPT, RPT)])


_deg_kernel = pl.kernel(
    _deg_body,
    out_type=jax.ShapeDtypeStruct((2, NP, HALF), jnp.float32),
    mesh=_mesh(),
    scratch_types=[
        pltpu.VMEM((NCHUNK, CHUNK), jnp.int32),
        pltpu.VMEM((CHUNK, HALF), jnp.float32),
        pltpu.VMEM_SHARED((NP, HALF), jnp.float32),
        pltpu.SemaphoreType.DMA,
    ],
)


def _agg_body(hflat_hbm, srcf_hbm, dstf_hbm, srco_hbm, dsto_hbm, zeros_hbm,
              outf_hbm, outo_hbm, src_v, dst_v, g0_v, g1_v, acc,
              sa0, sa1, sb0, sb1, ss0, ss1):
    c = lax.axis_index("c")
    t = lax.axis_index("s")
    HC = CHUNK // 2
    bufs = (g0_v, g1_v)
    asem = (sa0, sa1)
    bsem = (sb0, sb1)
    ssem = (ss0, ss1)

    def gather_start(j, b):
        pltpu.make_async_copy(
            hflat_hbm.at[src_v.at[j, pl.ds(0, HC)]],
            bufs[b].at[pl.ds(0, HC)], asem[b]).start()
        pltpu.make_async_copy(
            hflat_hbm.at[src_v.at[j, pl.ds(HC, HC)]],
            bufs[b].at[pl.ds(HC, HC)], bsem[b]).start()

    def gather_wait(j, b):
        pltpu.make_async_copy(
            hflat_hbm.at[src_v.at[j, pl.ds(0, HC)]],
            bufs[b].at[pl.ds(0, HC)], asem[b]).wait()
        pltpu.make_async_copy(
            hflat_hbm.at[src_v.at[j, pl.ds(HC, HC)]],
            bufs[b].at[pl.ds(HC, HC)], bsem[b]).wait()

    def scatter(j, b):
        return pltpu.make_async_copy(bufs[b], acc.at[dst_v.at[j]], ssem[b])

    for src_hbm, dst_hbm, out_hbm in ((srcf_hbm, dstf_hbm, outf_hbm),
                                      (srco_hbm, dsto_hbm, outo_hbm)):
        pltpu.sync_copy(zeros_hbm, acc.at[pl.ds(t * RPT, RPT)])
        pltpu.sync_copy(src_hbm.at[c, t], src_v)
        pltpu.sync_copy(dst_hbm.at[c, t], dst_v)
        plsc.subcore_barrier()

        gather_start(0, 0)

        def body(i, carry):
            j0 = 2 * i
            j1 = j0 + 1

            @pl.when(i > 0)
            def _():
                scatter(j0 - 1, 1).wait()

            gather_start(j1, 1)
            gather_wait(j0, 0)
            sc0 = scatter(j0, 0)
            sc0.start(add=True)
            sc0.wait()

            @pl.when(i < NCHUNK // 2 - 1)
            def _():
                gather_start(j0 + 2, 0)

            gather_wait(j1, 1)
            scatter(j1, 1).start(add=True)
            return carry

        lax.fori_loop(0, NCHUNK // 2, body, 0)
        scatter(NCHUNK - 1, 1).wait()
        plsc.subcore_barrier()
        pltpu.sync_copy(acc.at[pl.ds(t * RPT, RPT)],
                        out_hbm.at[c, pl.ds(t * RPT, RPT)])
        plsc.subcore_barrier()


_agg_kernel = pl.kernel(
    _agg_body,
    out_type=(jax.ShapeDtypeStruct((2, NP, HALF), jnp.float32),
              jax.ShapeDtypeStruct((2, NP, HALF), jnp.float32)),
    mesh=_mesh(),
    scratch_types=(
        [pltpu.VMEM((NCHUNK, CHUNK), jnp.int32),
         pltpu.VMEM((NCHUNK, CHUNK), jnp.int32)]
        + [pltpu.VMEM((CHUNK, HALF), jnp.float32)] * 2
        + [pltpu.VMEM_SHARED((NP, HALF), jnp.float32)]
        + [pltpu.SemaphoreType.DMA] * 6
    ),
)


BLK = 1024
GRID = NP // BLK


def _embed_body(x_ref, w_ref, b_ref, out_ref):
    r = jnp.dot(x_ref[...], w_ref[...],
                preferred_element_type=jnp.float32) + b_ref[...]
    out_ref[0, :, :] = r[:, :HALF]
    out_ref[1, :, :] = r[:, HALF:]


def _combine_body(h_ref, sf_ref, so_ref, deg_ref, ws_ref, wf_ref, wo_ref,
                  b_ref, out_ref):
    dot = functools.partial(jnp.dot, preferred_element_type=jnp.float32)
    invf = 1.0 / jnp.maximum(deg_ref[0, :, 0:1], 1.0)
    invo = 1.0 / jnp.maximum(deg_ref[1, :, 0:1], 1.0)
    acc = dot(h_ref[0], ws_ref[:HALF, :]) + dot(h_ref[1], ws_ref[HALF:, :])
    acc += dot(sf_ref[0] * invf, wf_ref[:HALF, :])
    acc += dot(sf_ref[1] * invf, wf_ref[HALF:, :])
    acc += dot(so_ref[0] * invo, wo_ref[:HALF, :])
    acc += dot(so_ref[1] * invo, wo_ref[HALF:, :])
    r = jnp.maximum(acc + b_ref[...], 0.0)
    out_ref[0, :, :] = r[:, :HALF]
    out_ref[1, :, :] = r[:, HALF:]


def _pool_body(h_ref, ws_ref, bs_ref, wp_ref, bp_ref, s1_ref, x1_ref):
    dot = functools.partial(jnp.dot, preferred_element_type=jnp.float32)
    logits = dot(h_ref[0], ws_ref[:HALF, :]) + dot(h_ref[1], ws_ref[HALF:, :])
    logits += bs_ref[...]
    s = jax.nn.softmax(logits, axis=-1)
    p = dot(h_ref[0], wp_ref[:HALF, :]) + dot(h_ref[1], wp_ref[HALF:, :])
    p += bp_ref[...]
    s1_ref[...] = s
    row = (lax.broadcasted_iota(jnp.int32, (BLK, 1), 0)
           + pl.program_id(0) * BLK)
    s = jnp.where(row < N, s, 0.0)
    part = lax.dot_general(s, p, (((0,), (0,)), ((), ())),
                           preferred_element_type=jnp.float32)

    @pl.when(pl.program_id(0) == 0)
    def _():
        x1_ref[...] = jnp.zeros_like(x1_ref)

    x1_ref[...] += part


def _cls_body(h_ref, s1_ref, x1_ref, wc1_ref, bc1_ref, wc2_ref, bc2_ref,
              out_ref):
    dot = functools.partial(jnp.dot, preferred_element_type=jnp.float32)
    x1 = x1_ref[...]
    pooled = jnp.mean(x1, axis=1)[None, :]
    prow = dot(pooled, wc1_ref[D:D + DN, :])
    sm = dot(s1_ref[...], x1)
    t = dot(h_ref[0], wc1_ref[:HALF, :]) + dot(h_ref[1], wc1_ref[HALF:D, :])
    t += dot(sm, wc1_ref[D + DN:, :]) + prow + bc1_ref[...]
    out_ref[...] = dot(t, wc2_ref[...]) + bc2_ref[...]


def _node_spec(cols):
    return pl.BlockSpec((2, BLK, cols), lambda i: (0, i, 0))


def _full(shape):
    return pl.BlockSpec(shape, lambda i: tuple(0 for _ in shape))


def _pad_deg(idx):
    pad = jnp.full((EPAD - E,), TRASH, jnp.int32)
    return jnp.concatenate([idx.astype(jnp.int32), pad]).reshape(
        NS, NCHUNK, CHUNK)


def _pad_agg_dst(idx):
    pad = jnp.full((EPAD - E,), TRASH, jnp.int32)
    d = jnp.concatenate([idx.astype(jnp.int32), pad]).reshape(
        NS, NCHUNK, CHUNK)
    return jnp.stack([d, d])


def _pad_agg_src(idx):
    pad = jnp.zeros((EPAD - E,), jnp.int32)
    s = jnp.concatenate([idx.astype(jnp.int32), pad]).reshape(
        NS, NCHUNK, CHUNK)
    return jnp.stack([s, s + NP])


def kernel(x, edge_index_forward, edge_index_onset, W_embed, b_embed,
           W0_fwd, W0_ons, W0_self, b0, W1_fwd, W1_ons, W1_self, b1,
           W_s, b_s, W_p, b_p, W_c1, b_c1, W_c2, b_c2):
    f32 = jnp.float32
    srcf = _pad_agg_src(edge_index_forward[0])
    dstf = _pad_agg_dst(edge_index_forward[1])
    srco = _pad_agg_src(edge_index_onset[0])
    dsto = _pad_agg_dst(edge_index_onset[1])
    dstT = jnp.stack([_pad_deg(edge_index_forward[1]),
                      _pad_deg(edge_index_onset[1])])

    zeros128 = jnp.zeros((RPT, HALF), f32)
    ones128 = jnp.ones((CHUNK, HALF), f32)


    deg = _deg_kernel(dstT, zeros128, ones128)

    x_pad = jnp.pad(x, ((0, NP - N), (0, HALF - F)))
    W_embed_pad = jnp.pad(W_embed, ((0, HALF - F), (0, 0)))

    h = pl.pallas_call(
        _embed_body,
        grid=(GRID,),
        in_specs=[pl.BlockSpec((BLK, HALF), lambda i: (i, 0)),
                  _full((HALF, D)), _full((1, D))],
        out_specs=_node_spec(HALF),
        out_shape=jax.ShapeDtypeStruct((2, NP, HALF), f32),
    )(x_pad, W_embed_pad, b_embed[None, :])

    def layer(h, Wf, Wo, Ws, b):
        hflat = h.reshape(2 * NP, HALF)
        sf, so = _agg_kernel(hflat, srcf, dstf, srco, dsto, zeros128)
        return pl.pallas_call(
            _combine_body,
            grid=(GRID,),
            in_specs=[_node_spec(HALF), _node_spec(HALF), _node_spec(HALF),
                      _node_spec(HALF),
                      _full((D, D)), _full((D, D)), _full((D, D)),
                      _full((1, D))],
            out_specs=_node_spec(HALF),
            out_shape=jax.ShapeDtypeStruct((2, NP, HALF), f32),
        )(h, sf, so, deg, Ws, Wf, Wo, b[None, :])

    h = layer(h, W0_fwd, W0_ons, W0_self, b0)
    h = layer(h, W1_fwd, W1_ons, W1_self, b1)

    s1, x1 = pl.pallas_call(
        _pool_body,
        grid=(GRID,),
        in_specs=[_node_spec(HALF), _full((D, DN)), _full((1, DN)),
                  _full((D, DD)), _full((1, DD))],
        out_specs=[pl.BlockSpec((BLK, DN), lambda i: (i, 0)),
                   _full((DN, DD))],
        out_shape=[jax.ShapeDtypeStruct((NP, DN), f32),
                   jax.ShapeDtypeStruct((DN, DD), f32)],
    )(h, W_s, b_s[None, :], W_p, b_p[None, :])

    out = pl.pallas_call(
        _cls_body,
        grid=(GRID,),
        in_specs=[_node_spec(HALF),
                  pl.BlockSpec((BLK, DN), lambda i: (i, 0)),
                  _full((DN, DD)),
                  _full((D + DN + DD, D)), _full((1, D)),
                  _full((D, D)), _full((1, D))],
        out_specs=pl.BlockSpec((BLK, D), lambda i: (i, 0)),
        out_shape=jax.ShapeDtypeStruct((NP, D), f32),
    )(h, s1, x1, W_c1, b_c1[None, :], W_c2, b_c2[None, :])
    return out[:N]


# speedup vs baseline: 1.1582x; 1.1582x over previous
import functools

import jax
import jax.numpy as jnp
from jax import lax
from jax.experimental import pallas as pl
from jax.experimental.pallas import tpu as pltpu
from jax.experimental.pallas import tpu_sc as plsc

N = 10000
NP = 10240
F = 111
D = 256
HALF = 128
DN = 32
DD = 64
E = 80000

NC = 2
NS = 16
CHUNK = 128
NCHUNK = 40
EPAD = NS * NCHUNK * CHUNK
TRASH = 10200
RPT = NP // NS

_mesh = functools.partial(
    plsc.VectorSubcoreMesh, core_axis_name="c", subcore_axis_name="s",
    num_cores=NC, num_subcores=NS)


def _deg_body(dst_hbm, zeros_hbm, ones_hbm, out_hbm, idx_v, ones_v, acc, sem):
    c = lax.axis_index("c")
    t = lax.axis_index("s")
    pltpu.sync_copy(zeros_hbm, acc.at[pl.ds(t * RPT, RPT)])
    pltpu.sync_copy(ones_hbm, ones_v)
    pltpu.sync_copy(dst_hbm.at[c, t], idx_v)
    plsc.subcore_barrier()

    def body(j, carry):
        pltpu.sync_copy(ones_v, acc.at[idx_v.at[j]], add=True)
        return carry

    lax.fori_loop(0, NCHUNK, body, 0)
    plsc.subcore_barrier()
    pltpu.sync_copy(acc.at[pl.ds(t * RPT, RPT)],
                    out_hbm.at[c, pl.ds(t * RPT, RPT)])


_deg_kernel = pl.kernel(
    _deg_body,
    out_type=jax.ShapeDtypeStruct((2, NP, HALF), jnp.float32),
    mesh=_mesh(),
    scratch_types=[
        pltpu.VMEM((NCHUNK, CHUNK), jnp.int32),
        pltpu.VMEM((CHUNK, HALF), jnp.float32),
        pltpu.VMEM_SHARED((NP, HALF), jnp.float32),
        pltpu.SemaphoreType.DMA,
    ],
)


def _agg_body(hflat_hbm, srcf_hbm, dstf_hbm, srco_hbm, dsto_hbm, zeros_hbm,
              outf_hbm, outo_hbm, src_v, dst_v, g0_v, g1_v, acc,
              sg0, sg1, ss0, ss1):
    c = lax.axis_index("c")
    t = lax.axis_index("s")

    def gather(j, buf, sem):
        return pltpu.make_async_copy(hflat_hbm.at[src_v.at[j]], buf, sem)

    def scatter(j, buf, sem):
        return pltpu.make_async_copy(buf, acc.at[dst_v.at[j]], sem)

    for src_hbm, dst_hbm, out_hbm in ((srcf_hbm, dstf_hbm, outf_hbm),
                                      (srco_hbm, dsto_hbm, outo_hbm)):
        pltpu.sync_copy(zeros_hbm, acc.at[pl.ds(t * RPT, RPT)])
        pltpu.sync_copy(src_hbm.at[c, t], src_v)
        pltpu.sync_copy(dst_hbm.at[c, t], dst_v)
        plsc.subcore_barrier()

        gather(0, g0_v, sg0).start()

        def body(i, carry):
            j0 = 2 * i
            j1 = j0 + 1

            @pl.when(i > 0)
            def _():
                scatter(j0 - 1, g1_v, ss1).wait()

            gather(j1, g1_v, sg1).start()
            gather(j0, g0_v, sg0).wait()
            sc0 = scatter(j0, g0_v, ss0)
            sc0.start(add=True)
            sc0.wait()

            @pl.when(i < NCHUNK // 2 - 1)
            def _():
                gather(j0 + 2, g0_v, sg0).start()

            gather(j1, g1_v, sg1).wait()
            scatter(j1, g1_v, ss1).start(add=True)
            return carry

        lax.fori_loop(0, NCHUNK // 2, body, 0)
        scatter(NCHUNK - 1, g1_v, ss1).wait()
        plsc.subcore_barrier()
        pltpu.sync_copy(acc.at[pl.ds(t * RPT, RPT)],
                        out_hbm.at[c, pl.ds(t * RPT, RPT)])
        plsc.subcore_barrier()


_agg_kernel = pl.kernel(
    _agg_body,
    out_type=(jax.ShapeDtypeStruct((2, NP, HALF), jnp.float32),
              jax.ShapeDtypeStruct((2, NP, HALF), jnp.float32)),
    mesh=_mesh(),
    scratch_types=(
        [pltpu.VMEM((NCHUNK, CHUNK), jnp.int32),
         pltpu.VMEM((NCHUNK, CHUNK), jnp.int32)]
        + [pltpu.VMEM((CHUNK, HALF), jnp.float32)] * 2
        + [pltpu.VMEM_SHARED((NP, HALF), jnp.float32)]
        + [pltpu.SemaphoreType.DMA] * 4
    ),
)


BLK = 1024
GRID = NP // BLK


def _embed_body(x_ref, w_ref, b_ref, out_ref):
    r = jnp.dot(x_ref[...], w_ref[...],
                preferred_element_type=jnp.float32) + b_ref[...]
    out_ref[0, :, :] = r[:, :HALF]
    out_ref[1, :, :] = r[:, HALF:]


def _mp_combine(h_ref, sf_ref, so_ref, deg_ref, ws_ref, wf_ref, wo_ref,
                b_ref):
    dot = functools.partial(jnp.dot, preferred_element_type=jnp.float32)
    invf = 1.0 / jnp.maximum(deg_ref[0, :, 0:1], 1.0)
    invo = 1.0 / jnp.maximum(deg_ref[1, :, 0:1], 1.0)
    acc = dot(h_ref[0], ws_ref[:HALF, :]) + dot(h_ref[1], ws_ref[HALF:, :])
    acc += dot(sf_ref[0] * invf, wf_ref[:HALF, :])
    acc += dot(sf_ref[1] * invf, wf_ref[HALF:, :])
    acc += dot(so_ref[0] * invo, wo_ref[:HALF, :])
    acc += dot(so_ref[1] * invo, wo_ref[HALF:, :])
    return jnp.maximum(acc + b_ref[...], 0.0)


def _combine_body(h_ref, sf_ref, so_ref, deg_ref, ws_ref, wf_ref, wo_ref,
                  b_ref, out_ref):
    r = _mp_combine(h_ref, sf_ref, so_ref, deg_ref, ws_ref, wf_ref, wo_ref,
                    b_ref)
    out_ref[0, :, :] = r[:, :HALF]
    out_ref[1, :, :] = r[:, HALF:]


def _combine_pool_body(h_ref, sf_ref, so_ref, deg_ref, ws_ref, wf_ref,
                       wo_ref, b_ref, wsm_ref, bs_ref, wp_ref, bp_ref,
                       out_ref, s1_ref, x1_ref):
    dot = functools.partial(jnp.dot, preferred_element_type=jnp.float32)
    r = _mp_combine(h_ref, sf_ref, so_ref, deg_ref, ws_ref, wf_ref, wo_ref,
                    b_ref)
    out_ref[0, :, :] = r[:, :HALF]
    out_ref[1, :, :] = r[:, HALF:]
    logits = dot(r, wsm_ref[...]) + bs_ref[...]
    s = jax.nn.softmax(logits, axis=-1)
    p = dot(r, wp_ref[...]) + bp_ref[...]
    s1_ref[...] = s
    row = (lax.broadcasted_iota(jnp.int32, (BLK, 1), 0)
           + pl.program_id(0) * BLK)
    s = jnp.where(row < N, s, 0.0)
    part = lax.dot_general(s, p, (((0,), (0,)), ((), ())),
                           preferred_element_type=jnp.float32)

    @pl.when(pl.program_id(0) == 0)
    def _():
        x1_ref[...] = jnp.zeros_like(x1_ref)

    x1_ref[...] += part


def _cls_body(h_ref, s1_ref, x1_ref, wc1_ref, bc1_ref, wc2_ref, bc2_ref,
              out_ref):
    dot = functools.partial(jnp.dot, preferred_element_type=jnp.float32)
    x1 = x1_ref[...]
    pooled = jnp.mean(x1, axis=1)[None, :]
    prow = dot(pooled, wc1_ref[D:D + DN, :])
    sm = dot(s1_ref[...], x1)
    t = dot(h_ref[0], wc1_ref[:HALF, :]) + dot(h_ref[1], wc1_ref[HALF:D, :])
    t += dot(sm, wc1_ref[D + DN:, :]) + prow + bc1_ref[...]
    out_ref[...] = dot(t, wc2_ref[...]) + bc2_ref[...]


def _node_spec(cols):
    return pl.BlockSpec((2, BLK, cols), lambda i: (0, i, 0))


def _full(shape):
    return pl.BlockSpec(shape, lambda i: tuple(0 for _ in shape))


def _pad_deg(idx):
    pad = jnp.full((EPAD - E,), TRASH, jnp.int32)
    return jnp.concatenate([idx.astype(jnp.int32), pad]).reshape(
        NS, NCHUNK, CHUNK)


def _pad_agg_dst(idx):
    pad = jnp.full((EPAD - E,), TRASH, jnp.int32)
    d = jnp.concatenate([idx.astype(jnp.int32), pad]).reshape(
        NS, NCHUNK, CHUNK)
    return jnp.stack([d, d])


def _pad_agg_src(idx):
    pad = jnp.zeros((EPAD - E,), jnp.int32)
    s = jnp.concatenate([idx.astype(jnp.int32), pad]).reshape(
        NS, NCHUNK, CHUNK)
    return jnp.stack([s, s + NP])


def kernel(x, edge_index_forward, edge_index_onset, W_embed, b_embed,
           W0_fwd, W0_ons, W0_self, b0, W1_fwd, W1_ons, W1_self, b1,
           W_s, b_s, W_p, b_p, W_c1, b_c1, W_c2, b_c2):
    f32 = jnp.float32
    srcf = _pad_agg_src(edge_index_forward[0])
    dstf = _pad_agg_dst(edge_index_forward[1])
    srco = _pad_agg_src(edge_index_onset[0])
    dsto = _pad_agg_dst(edge_index_onset[1])
    dstT = jnp.stack([_pad_deg(edge_index_forward[1]),
                      _pad_deg(edge_index_onset[1])])

    zeros128 = jnp.zeros((RPT, HALF), f32)
    ones128 = jnp.ones((CHUNK, HALF), f32)


    deg = _deg_kernel(dstT, zeros128, ones128)

    x_pad = jnp.pad(x, ((0, NP - N), (0, HALF - F)))
    W_embed_pad = jnp.pad(W_embed, ((0, HALF - F), (0, 0)))

    h = pl.pallas_call(
        _embed_body,
        grid=(GRID,),
        in_specs=[pl.BlockSpec((BLK, HALF), lambda i: (i, 0)),
                  _full((HALF, D)), _full((1, D))],
        out_specs=_node_spec(HALF),
        out_shape=jax.ShapeDtypeStruct((2, NP, HALF), f32),
    )(x_pad, W_embed_pad, b_embed[None, :])

    hflat = h.reshape(2 * NP, HALF)
    sf, so = _agg_kernel(hflat, srcf, dstf, srco, dsto, zeros128)
    h = pl.pallas_call(
        _combine_body,
        grid=(GRID,),
        in_specs=[_node_spec(HALF), _node_spec(HALF), _node_spec(HALF),
                  _node_spec(HALF),
                  _full((D, D)), _full((D, D)), _full((D, D)),
                  _full((1, D))],
        out_specs=_node_spec(HALF),
        out_shape=jax.ShapeDtypeStruct((2, NP, HALF), f32),
    )(h, sf, so, deg, W0_self, W0_fwd, W0_ons, b0[None, :])

    hflat = h.reshape(2 * NP, HALF)
    sf, so = _agg_kernel(hflat, srcf, dstf, srco, dsto, zeros128)
    h, s1, x1 = pl.pallas_call(
        _combine_pool_body,
        grid=(GRID,),
        in_specs=[_node_spec(HALF), _node_spec(HALF), _node_spec(HALF),
                  _node_spec(HALF),
                  _full((D, D)), _full((D, D)), _full((D, D)),
                  _full((1, D)),
                  _full((D, DN)), _full((1, DN)),
                  _full((D, DD)), _full((1, DD))],
        out_specs=[_node_spec(HALF),
                   pl.BlockSpec((BLK, DN), lambda i: (i, 0)),
                   _full((DN, DD))],
        out_shape=[jax.ShapeDtypeStruct((2, NP, HALF), f32),
                   jax.ShapeDtypeStruct((NP, DN), f32),
                   jax.ShapeDtypeStruct((DN, DD), f32)],
    )(h, sf, so, deg, W1_self, W1_fwd, W1_ons, b1[None, :],
      W_s, b_s[None, :], W_p, b_p[None, :])

    out = pl.pallas_call(
        _cls_body,
        grid=(GRID,),
        in_specs=[_node_spec(HALF),
                  pl.BlockSpec((BLK, DN), lambda i: (i, 0)),
                  _full((DN, DD)),
                  _full((D + DN + DD, D)), _full((1, D)),
                  _full((D, D)), _full((1, D))],
        out_specs=pl.BlockSpec((BLK, D), lambda i: (i, 0)),
        out_shape=jax.ShapeDtypeStruct((NP, D), f32),
    )(h, s1, x1, W_c1, b_c1[None, :], W_c2, b_c2[None, :])
    return out[:N]


# speedup vs baseline: 1.1658x; 1.0066x over previous
import functools

import jax
import jax.numpy as jnp
from jax import lax
from jax.experimental import pallas as pl
from jax.experimental.pallas import tpu as pltpu
from jax.experimental.pallas import tpu_sc as plsc

N = 10000
NP = 10240
F = 111
D = 256
HALF = 128
DN = 32
DD = 64
E = 80000

NC = 2
NS = 16
CHUNK = 128
NCHUNK = 40
EPAD = NS * NCHUNK * CHUNK
TRASH = 10200
RPT = NP // NS

_mesh = functools.partial(
    plsc.VectorSubcoreMesh, core_axis_name="c", subcore_axis_name="s",
    num_cores=NC, num_subcores=NS)


def _deg_body(dst_hbm, zeros_hbm, ones_hbm, out_hbm, idx_v, ones_v, acc, sem):
    c = lax.axis_index("c")
    t = lax.axis_index("s")
    pltpu.sync_copy(zeros_hbm, acc.at[pl.ds(t * RPT, RPT)])
    pltpu.sync_copy(ones_hbm, ones_v)
    pltpu.sync_copy(dst_hbm.at[c, t], idx_v)
    plsc.subcore_barrier()

    def body(j, carry):
        pltpu.sync_copy(ones_v, acc.at[idx_v.at[j]], add=True)
        return carry

    lax.fori_loop(0, NCHUNK, body, 0)
    plsc.subcore_barrier()
    pltpu.sync_copy(acc.at[pl.ds(t * RPT, RPT)],
                    out_hbm.at[c, pl.ds(t * RPT, RPT)])


_deg_kernel = pl.kernel(
    _deg_body,
    out_type=jax.ShapeDtypeStruct((2, NP, HALF), jnp.float32),
    mesh=_mesh(),
    scratch_types=[
        pltpu.VMEM((NCHUNK, CHUNK), jnp.int32),
        pltpu.VMEM((CHUNK, HALF), jnp.float32),
        pltpu.VMEM_SHARED((NP, HALF), jnp.float32),
        pltpu.SemaphoreType.DMA,
    ],
)


def _agg_body(hflat_hbm, srcf_hbm, dstf_hbm, srco_hbm, dsto_hbm, zeros_hbm,
              outf_hbm, outo_hbm, src_v, dst_v, g0_v, g1_v, acc,
              sg0, sg1, ss0, ss1):
    c = lax.axis_index("c")
    t = lax.axis_index("s")

    def gather(j, buf, sem):
        return pltpu.make_async_copy(hflat_hbm.at[src_v.at[j]], buf, sem)

    def scatter(j, buf, sem):
        return pltpu.make_async_copy(buf, acc.at[dst_v.at[j]], sem)

    for src_hbm, dst_hbm, out_hbm in ((srcf_hbm, dstf_hbm, outf_hbm),
                                      (srco_hbm, dsto_hbm, outo_hbm)):
        pltpu.sync_copy(zeros_hbm, acc.at[pl.ds(t * RPT, RPT)])
        pltpu.sync_copy(src_hbm.at[c, t], src_v)
        pltpu.sync_copy(dst_hbm.at[c, t], dst_v)
        plsc.subcore_barrier()

        gather(0, g0_v, sg0).start()

        def body(i, carry):
            j0 = 2 * i
            j1 = j0 + 1

            @pl.when(i > 0)
            def _():
                scatter(j0 - 1, g1_v, ss1).wait()

            gather(j1, g1_v, sg1).start()
            gather(j0, g0_v, sg0).wait()
            sc0 = scatter(j0, g0_v, ss0)
            sc0.start(add=True)
            sc0.wait()

            @pl.when(i < NCHUNK // 2 - 1)
            def _():
                gather(j0 + 2, g0_v, sg0).start()

            gather(j1, g1_v, sg1).wait()
            scatter(j1, g1_v, ss1).start(add=True)
            return carry

        lax.fori_loop(0, NCHUNK // 2, body, 0)
        scatter(NCHUNK - 1, g1_v, ss1).wait()
        plsc.subcore_barrier()
        pltpu.sync_copy(acc.at[pl.ds(t * RPT, RPT)],
                        out_hbm.at[c, pl.ds(t * RPT, RPT)])
        plsc.subcore_barrier()


_agg_kernel = pl.kernel(
    _agg_body,
    out_type=(jax.ShapeDtypeStruct((2, NP, HALF), jnp.float32),
              jax.ShapeDtypeStruct((2, NP, HALF), jnp.float32)),
    mesh=_mesh(),
    scratch_types=(
        [pltpu.VMEM((NCHUNK, CHUNK), jnp.int32),
         pltpu.VMEM((NCHUNK, CHUNK), jnp.int32)]
        + [pltpu.VMEM((CHUNK, HALF), jnp.float32)] * 2
        + [pltpu.VMEM_SHARED((NP, HALF), jnp.float32)]
        + [pltpu.SemaphoreType.DMA] * 4
    ),
)


BLK = 2048
GRID = NP // BLK


def _embed_body(x_ref, w_ref, b_ref, out_ref):
    r = jnp.dot(x_ref[...], w_ref[...],
                preferred_element_type=jnp.float32) + b_ref[...]
    out_ref[0, :, :] = r[:, :HALF]
    out_ref[1, :, :] = r[:, HALF:]


def _mp_combine(h_ref, sf_ref, so_ref, deg_ref, ws_ref, wf_ref, wo_ref,
                b_ref):
    dot = functools.partial(jnp.dot, preferred_element_type=jnp.float32)
    invf = 1.0 / jnp.maximum(deg_ref[0, :, 0:1], 1.0)
    invo = 1.0 / jnp.maximum(deg_ref[1, :, 0:1], 1.0)
    acc = dot(h_ref[0], ws_ref[:HALF, :]) + dot(h_ref[1], ws_ref[HALF:, :])
    acc += dot(sf_ref[0] * invf, wf_ref[:HALF, :])
    acc += dot(sf_ref[1] * invf, wf_ref[HALF:, :])
    acc += dot(so_ref[0] * invo, wo_ref[:HALF, :])
    acc += dot(so_ref[1] * invo, wo_ref[HALF:, :])
    return jnp.maximum(acc + b_ref[...], 0.0)


def _combine_body(h_ref, sf_ref, so_ref, deg_ref, ws_ref, wf_ref, wo_ref,
                  b_ref, out_ref):
    r = _mp_combine(h_ref, sf_ref, so_ref, deg_ref, ws_ref, wf_ref, wo_ref,
                    b_ref)
    out_ref[0, :, :] = r[:, :HALF]
    out_ref[1, :, :] = r[:, HALF:]


def _combine_pool_body(h_ref, sf_ref, so_ref, deg_ref, ws_ref, wf_ref,
                       wo_ref, b_ref, wsm_ref, bs_ref, wp_ref, bp_ref,
                       out_ref, s1_ref, x1_ref):
    dot = functools.partial(jnp.dot, preferred_element_type=jnp.float32)
    r = _mp_combine(h_ref, sf_ref, so_ref, deg_ref, ws_ref, wf_ref, wo_ref,
                    b_ref)
    out_ref[0, :, :] = r[:, :HALF]
    out_ref[1, :, :] = r[:, HALF:]
    logits = dot(r, wsm_ref[...]) + bs_ref[...]
    s = jax.nn.softmax(logits, axis=-1)
    p = dot(r, wp_ref[...]) + bp_ref[...]
    s1_ref[...] = s
    row = (lax.broadcasted_iota(jnp.int32, (BLK, 1), 0)
           + pl.program_id(0) * BLK)
    s = jnp.where(row < N, s, 0.0)
    part = lax.dot_general(s, p, (((0,), (0,)), ((), ())),
                           preferred_element_type=jnp.float32)

    @pl.when(pl.program_id(0) == 0)
    def _():
        x1_ref[...] = jnp.zeros_like(x1_ref)

    x1_ref[...] += part


def _cls_body(h_ref, s1_ref, x1_ref, wc1_ref, bc1_ref, wc2_ref, bc2_ref,
              out_ref):
    dot = functools.partial(jnp.dot, preferred_element_type=jnp.float32)
    x1 = x1_ref[...]
    pooled = jnp.mean(x1, axis=1)[None, :]
    prow = dot(pooled, wc1_ref[D:D + DN, :])
    sm = dot(s1_ref[...], x1)
    t = dot(h_ref[0], wc1_ref[:HALF, :]) + dot(h_ref[1], wc1_ref[HALF:D, :])
    t += dot(sm, wc1_ref[D + DN:, :]) + prow + bc1_ref[...]
    out_ref[...] = dot(t, wc2_ref[...]) + bc2_ref[...]


def _node_spec(cols):
    return pl.BlockSpec((2, BLK, cols), lambda i: (0, i, 0))


def _full(shape):
    return pl.BlockSpec(shape, lambda i: tuple(0 for _ in shape))


def _pad_deg(idx):
    pad = jnp.full((EPAD - E,), TRASH, jnp.int32)
    return jnp.concatenate([idx.astype(jnp.int32), pad]).reshape(
        NS, NCHUNK, CHUNK)


def _pad_agg_dst(idx):
    pad = jnp.full((EPAD - E,), TRASH, jnp.int32)
    d = jnp.concatenate([idx.astype(jnp.int32), pad]).reshape(
        NS, NCHUNK, CHUNK)
    return jnp.stack([d, d])


def _pad_agg_src(idx):
    pad = jnp.zeros((EPAD - E,), jnp.int32)
    s = jnp.concatenate([idx.astype(jnp.int32), pad]).reshape(
        NS, NCHUNK, CHUNK)
    return jnp.stack([s, s + NP])


def kernel(x, edge_index_forward, edge_index_onset, W_embed, b_embed,
           W0_fwd, W0_ons, W0_self, b0, W1_fwd, W1_ons, W1_self, b1,
           W_s, b_s, W_p, b_p, W_c1, b_c1, W_c2, b_c2):
    f32 = jnp.float32
    srcf = _pad_agg_src(edge_index_forward[0])
    dstf = _pad_agg_dst(edge_index_forward[1])
    srco = _pad_agg_src(edge_index_onset[0])
    dsto = _pad_agg_dst(edge_index_onset[1])
    dstT = jnp.stack([_pad_deg(edge_index_forward[1]),
                      _pad_deg(edge_index_onset[1])])

    zeros128 = jnp.zeros((RPT, HALF), f32)
    ones128 = jnp.ones((CHUNK, HALF), f32)


    deg = _deg_kernel(dstT, zeros128, ones128)

    x_pad = jnp.pad(x, ((0, NP - N), (0, HALF - F)))
    W_embed_pad = jnp.pad(W_embed, ((0, HALF - F), (0, 0)))

    h = pl.pallas_call(
        _embed_body,
        grid=(GRID,),
        in_specs=[pl.BlockSpec((BLK, HALF), lambda i: (i, 0)),
                  _full((HALF, D)), _full((1, D))],
        out_specs=_node_spec(HALF),
        out_shape=jax.ShapeDtypeStruct((2, NP, HALF), f32),
    )(x_pad, W_embed_pad, b_embed[None, :])

    hflat = h.reshape(2 * NP, HALF)
    sf, so = _agg_kernel(hflat, srcf, dstf, srco, dsto, zeros128)
    h = pl.pallas_call(
        _combine_body,
        grid=(GRID,),
        in_specs=[_node_spec(HALF), _node_spec(HALF), _node_spec(HALF),
                  _node_spec(HALF),
                  _full((D, D)), _full((D, D)), _full((D, D)),
                  _full((1, D))],
        out_specs=_node_spec(HALF),
        out_shape=jax.ShapeDtypeStruct((2, NP, HALF), f32),
    )(h, sf, so, deg, W0_self, W0_fwd, W0_ons, b0[None, :])

    hflat = h.reshape(2 * NP, HALF)
    sf, so = _agg_kernel(hflat, srcf, dstf, srco, dsto, zeros128)
    h, s1, x1 = pl.pallas_call(
        _combine_pool_body,
        grid=(GRID,),
        in_specs=[_node_spec(HALF), _node_spec(HALF), _node_spec(HALF),
                  _node_spec(HALF),
                  _full((D, D)), _full((D, D)), _full((D, D)),
                  _full((1, D)),
                  _full((D, DN)), _full((1, DN)),
                  _full((D, DD)), _full((1, DD))],
        out_specs=[_node_spec(HALF),
                   pl.BlockSpec((BLK, DN), lambda i: (i, 0)),
                   _full((DN, DD))],
        out_shape=[jax.ShapeDtypeStruct((2, NP, HALF), f32),
                   jax.ShapeDtypeStruct((NP, DN), f32),
                   jax.ShapeDtypeStruct((DN, DD), f32)],
    )(h, sf, so, deg, W1_self, W1_fwd, W1_ons, b1[None, :],
      W_s, b_s[None, :], W_p, b_p[None, :])

    out = pl.pallas_call(
        _cls_body,
        grid=(GRID,),
        in_specs=[_node_spec(HALF),
                  pl.BlockSpec((BLK, DN), lambda i: (i, 0)),
                  _full((DN, DD)),
                  _full((D + DN + DD, D)), _full((1, D)),
                  _full((D, D)), _full((1, D))],
        out_specs=pl.BlockSpec((BLK, D), lambda i: (i, 0)),
        out_shape=jax.ShapeDtypeStruct((NP, D), f32),
    )(h, s1, x1, W_c1, b_c1[None, :], W_c2, b_c2[None, :])
    return out[:N]
